# baseline (speedup 1.0000x reference)
import functools
import math

import jax
import jax.numpy as jnp
from jax import lax
from jax.experimental import pallas as pl
from jax.experimental.pallas import tpu as pltpu
from jax.experimental.pallas import tpu_sc as plsc

D_MODEL = 128
SEQ = 200
LANES = 16
NV = D_MODEL // LANES
NC, NS = 2, 16
NW = NC * NS
EPS = 1e-6
SQRTD = math.sqrt(D_MODEL)
IDX_W = 100
NBUF = 2
UNROLL = 3

_MAGIC = 0x5F3759DF


def _lane_sum(v):
    dnums = lax.GatherDimensionNumbers(
        offset_dims=(), collapsed_slice_dims=(0,), start_index_map=(0,))
    for k in (8, 4, 2, 1):
        perm = jnp.bitwise_xor(lax.iota(jnp.int32, LANES), jnp.int32(k))
        shuf = lax.gather(
            v, perm[:, None], dimension_numbers=dnums, slice_sizes=(1,),
            mode=lax.GatherScatterMode.PROMISE_IN_BOUNDS)
        v = v + shuf
    return v


def _ln_row(buf, pos_v, r, a_regs, b_regs):
    h = [
        buf[r, pl.ds(k * LANES, LANES)] + pos_v[r, pl.ds(k * LANES, LANES)]
        for k in range(NV)
    ]
    q = [h[k] * h[k] for k in range(NV)]
    s01, s23 = h[0] + h[1], h[2] + h[3]
    s45, s67 = h[4] + h[5], h[6] + h[7]
    sum_b = _lane_sum((s01 + s23) + (s45 + s67))
    q01, q23 = q[0] + q[1], q[2] + q[3]
    q45, q67 = q[4] + q[5], q[6] + q[7]
    ssq_b = _lane_sum((q01 + q23) + (q45 + q67))
    mean_b = sum_b * jnp.float32(1.0 / D_MODEL)
    var_b = (ssq_b - (mean_b * mean_b) * jnp.float32(D_MODEL)) * jnp.float32(
        1.0 / (D_MODEL - 1))
    var_b = jnp.maximum(var_b, jnp.float32(1e-12))
    yi = jnp.int32(_MAGIC) - lax.shift_right_logical(
        lax.bitcast_convert_type(var_b, jnp.int32), 1)
    y = lax.bitcast_convert_type(yi, jnp.float32)
    half_v = var_b * jnp.float32(0.5)
    for _ in range(1):
        y = y * (jnp.float32(1.5) - half_v * y * y)
    inv_b = y
    for k in range(NV):
        c = inv_b * a_regs[k]
        buf[r, pl.ds(k * LANES, LANES)] = (h[k] - mean_b) * c + b_regs[k]


def _make_kernel(batch):
    rpw = batch * SEQ // NW
    nchunk = rpw // SEQ
    nidx = rpw // IDX_W
    nloop = nchunk // NBUF
    assert nchunk % NBUF == 0
    mesh = plsc.VectorSubcoreMesh(core_axis_name="c", subcore_axis_name="s")

    @functools.partial(
        pl.kernel,
        out_type=jax.ShapeDtypeStruct((batch, SEQ, D_MODEL), jnp.float32),
        mesh=mesh,
        scratch_types=[
            pltpu.VMEM((nidx, IDX_W), jnp.int32),
            pltpu.VMEM((NBUF, SEQ, D_MODEL), jnp.float32),
            pltpu.VMEM((SEQ, D_MODEL), jnp.float32),
            pltpu.VMEM((D_MODEL,), jnp.float32),
            pltpu.VMEM((D_MODEL,), jnp.float32),
        ] + [pltpu.SemaphoreType.DMA] * (2 * NBUF),
    )
    def emb_ln(x_hbm, tok_hbm, pos_hbm, a_hbm, b_hbm, out_hbm,
               idx_v, rows_v, pos_v, a_v, b_v, *sems):
        gsems, ssems = sems[:NBUF], sems[NBUF:]
        wid = lax.axis_index("s") * NC + lax.axis_index("c")
        pltpu.sync_copy(pos_hbm, pos_v)
        pltpu.sync_copy(a_hbm, a_v)
        pltpu.sync_copy(b_hbm, b_v)
        pltpu.sync_copy(
            x_hbm.at[pl.ds(pl.multiple_of(wid * nidx, 8), nidx)], idx_v)
        a_regs = [
            a_v[pl.ds(k * LANES, LANES)] * jnp.float32(SQRTD) for k in range(NV)
        ]
        b_regs = [
            b_v[pl.ds(k * LANES, LANES)] * jnp.float32(SQRTD) for k in range(NV)
        ]
        seq0 = wid * nchunk

        def start_gather(k, j):
            pltpu.async_copy(tok_hbm.at[idx_v.at[2 * k]],
                             rows_v.at[j].at[pl.ds(0, IDX_W)], gsems[j])
            pltpu.async_copy(tok_hbm.at[idx_v.at[2 * k + 1]],
                             rows_v.at[j].at[pl.ds(IDX_W, IDX_W)], gsems[j])

        def wait_gather(j):
            for _ in range(2):
                pltpu.make_async_copy(
                    tok_hbm.at[idx_v.at[0]],
                    rows_v.at[j].at[pl.ds(0, IDX_W)], gsems[j]).wait()

        def start_scatter(k, j):
            pltpu.async_copy(rows_v.at[j], out_hbm.at[seq0 + k], ssems[j])

        def wait_scatter(j):
            pltpu.make_async_copy(rows_v.at[j], out_hbm.at[0], ssems[j]).wait()

        start_gather(0, 0)

        def compute_half(j, half):
            buf = rows_v.at[j]

            @plsc.parallel_loop(half * IDX_W, (half + 1) * IDX_W, step=1,
                                unroll=UNROLL)
            def _(r):
                _ln_row(buf, pos_v, r, a_regs, b_regs)

        def loop_body(i, carry):
            for j in range(NBUF):
                k = i * NBUF + j
                wait_gather(j)
                compute_half(j, 0)
                if j == 0:
                    @pl.when(i > 0)
                    def _():
                        wait_scatter(NBUF - 1)
                else:
                    wait_scatter(j - 1)
                if j < NBUF - 1:
                    start_gather(k + 1, j + 1)
                else:
                    @pl.when(i < nloop - 1)
                    def _(k=k):
                        start_gather(k + 1, 0)
                compute_half(j, 1)
                start_scatter(k, j)
            return carry

        lax.fori_loop(0, nloop, loop_body, 0)
        wait_scatter(NBUF - 1)

    return emb_ln


@jax.jit
def kernel(x, tok_table, pos_table, a, b):
    batch, seq = x.shape
    assert seq == SEQ and tok_table.shape[1] == D_MODEL
    x2d = x.reshape(batch * seq // IDX_W, IDX_W).astype(jnp.int32)
    return _make_kernel(batch)(x2d, tok_table, pos_table, a, b)


# speedup vs baseline: 1.2396x; 1.2396x over previous
import functools
import math

import jax
import jax.numpy as jnp
from jax import lax
from jax.experimental import pallas as pl
from jax.experimental.pallas import tpu as pltpu
from jax.experimental.pallas import tpu_sc as plsc

D_MODEL = 128
SEQ = 200
LANES = 16
NV = D_MODEL // LANES
NC, NS = 2, 16
NW = NC * NS
EPS = 1e-6
SQRTD = math.sqrt(D_MODEL)
IDX_W = 100
NBUF = 2
UNROLL = 2

_MAGIC = 0x5F3759DF


def _lane_sum(v):
    dnums = lax.GatherDimensionNumbers(
        offset_dims=(), collapsed_slice_dims=(0,), start_index_map=(0,))
    for k in (8, 4, 2, 1):
        perm = jnp.bitwise_xor(lax.iota(jnp.int32, LANES), jnp.int32(k))
        shuf = lax.gather(
            v, perm[:, None], dimension_numbers=dnums, slice_sizes=(1,),
            mode=lax.GatherScatterMode.PROMISE_IN_BOUNDS)
        v = v + shuf
    return v


def _ln_row(buf, pos_v, r, a_regs, b_regs):
    h = [
        buf[r, pl.ds(k * LANES, LANES)] + pos_v[r, pl.ds(k * LANES, LANES)]
        for k in range(NV)
    ]
    q = [h[k] * h[k] for k in range(NV)]
    s01, s23 = h[0] + h[1], h[2] + h[3]
    s45, s67 = h[4] + h[5], h[6] + h[7]
    sum_b = _lane_sum((s01 + s23) + (s45 + s67))
    q01, q23 = q[0] + q[1], q[2] + q[3]
    q45, q67 = q[4] + q[5], q[6] + q[7]
    ssq_b = _lane_sum((q01 + q23) + (q45 + q67))
    mean_b = sum_b * jnp.float32(1.0 / D_MODEL)
    var_b = (ssq_b - (mean_b * mean_b) * jnp.float32(D_MODEL)) * jnp.float32(
        1.0 / (D_MODEL - 1))
    var_b = jnp.maximum(var_b, jnp.float32(1e-12))
    yi = jnp.int32(_MAGIC) - lax.shift_right_logical(
        lax.bitcast_convert_type(var_b, jnp.int32), 1)
    y = lax.bitcast_convert_type(yi, jnp.float32)
    half_v = var_b * jnp.float32(0.5)
    for _ in range(1):
        y = y * (jnp.float32(1.5) - half_v * y * y)
    inv_b = y
    for k in range(NV):
        c = inv_b * a_regs[k]
        buf[r, pl.ds(k * LANES, LANES)] = (h[k] - mean_b) * c + b_regs[k]


def _make_kernel(batch):
    rpw = batch * SEQ // NW
    nchunk = rpw // SEQ
    nidx = rpw // IDX_W
    nloop = nchunk // NBUF
    assert nchunk % NBUF == 0
    mesh = plsc.VectorSubcoreMesh(core_axis_name="c", subcore_axis_name="s")

    @functools.partial(
        pl.kernel,
        out_type=jax.ShapeDtypeStruct((batch, SEQ, D_MODEL), jnp.float32),
        mesh=mesh,
        scratch_types=[
            pltpu.VMEM((nidx, IDX_W), jnp.int32),
            pltpu.VMEM((NBUF, SEQ, D_MODEL), jnp.float32),
            pltpu.VMEM((SEQ, D_MODEL), jnp.float32),
            pltpu.VMEM((D_MODEL,), jnp.float32),
            pltpu.VMEM((D_MODEL,), jnp.float32),
        ] + [pltpu.SemaphoreType.DMA] * (2 * NBUF),
    )
    def emb_ln(x_hbm, tok_hbm, pos_hbm, a_hbm, b_hbm, out_hbm,
               idx_v, rows_v, pos_v, a_v, b_v, *sems):
        gsems, ssems = sems[:NBUF], sems[NBUF:]
        wid = lax.axis_index("s") * NC + lax.axis_index("c")
        pltpu.sync_copy(pos_hbm, pos_v)
        pltpu.sync_copy(a_hbm, a_v)
        pltpu.sync_copy(b_hbm, b_v)
        pltpu.sync_copy(
            x_hbm.at[pl.ds(pl.multiple_of(wid * nidx, 8), nidx)], idx_v)
        a_regs = [
            a_v[pl.ds(k * LANES, LANES)] * jnp.float32(SQRTD) for k in range(NV)
        ]
        b_regs = [
            b_v[pl.ds(k * LANES, LANES)] * jnp.float32(SQRTD) for k in range(NV)
        ]
        seq0 = wid * nchunk

        def start_gather(k, j):
            pltpu.async_copy(tok_hbm.at[idx_v.at[2 * k]],
                             rows_v.at[j].at[pl.ds(0, IDX_W)], gsems[j])
            pltpu.async_copy(tok_hbm.at[idx_v.at[2 * k + 1]],
                             rows_v.at[j].at[pl.ds(IDX_W, IDX_W)], gsems[j])

        def wait_gather(j):
            for _ in range(2):
                pltpu.make_async_copy(
                    tok_hbm.at[idx_v.at[0]],
                    rows_v.at[j].at[pl.ds(0, IDX_W)], gsems[j]).wait()

        def start_scatter(k, j):
            pltpu.async_copy(rows_v.at[j], out_hbm.at[seq0 + k], ssems[j])

        def wait_scatter(j):
            pltpu.make_async_copy(rows_v.at[j], out_hbm.at[0], ssems[j]).wait()

        start_gather(0, 0)

        def compute_half(j, half):
            buf = rows_v.at[j]

            @plsc.parallel_loop(half * IDX_W, (half + 1) * IDX_W, step=1,
                                unroll=UNROLL)
            def _(r):
                _ln_row(buf, pos_v, r, a_regs, b_regs)

        def loop_body(i, carry):
            for j in range(NBUF):
                k = i * NBUF + j
                wait_gather(j)
                compute_half(j, 0)
                if j == 0:
                    @pl.when(i > 0)
                    def _():
                        wait_scatter(NBUF - 1)
                else:
                    wait_scatter(j - 1)
                if j < NBUF - 1:
                    start_gather(k + 1, j + 1)
                else:
                    @pl.when(i < nloop - 1)
                    def _(k=k):
                        start_gather(k + 1, 0)
                compute_half(j, 1)
                start_scatter(k, j)
            return carry

        lax.fori_loop(0, nloop, loop_body, 0)
        wait_scatter(NBUF - 1)

    return emb_ln


@jax.jit
def kernel(x, tok_table, pos_table, a, b):
    batch, seq = x.shape
    assert seq == SEQ and tok_table.shape[1] == D_MODEL
    x2d = x.reshape(batch * seq // IDX_W, IDX_W).astype(jnp.int32)
    return _make_kernel(batch)(x2d, tok_table, pos_table, a, b)
